# Initial kernel scaffold; baseline (speedup 1.0000x reference)
import jax
import jax.numpy as jnp
from jax.experimental import pallas as pl


def kernel(x, edge_index, W1, b1, W2, b2, W3, b3, g1, be1, g2, be2, Wc1a, bc1a, Wc1b, bc1b, Wc2a, bc2a, Wc2b, bc2b):
    raise NotImplementedError("write your pallas kernel here")


# speedup vs baseline: 10.0392x; 10.0392x over previous
import functools

import jax
import jax.numpy as jnp
from jax import lax
from jax.experimental import pallas as pl
from jax.experimental.pallas import tpu as pltpu
from jax.experimental.pallas import tpu_sc as plsc

N = 10000
NP = 10240
E = 320000
D = 128
NC = 2
NS = 16
EPT = E // (NC * NS)
CHUNK = 80
NCHUNK = EPT // CHUNK
RPT = NP // NS

_mesh = plsc.VectorSubcoreMesh(
    core_axis_name="c", subcore_axis_name="s", num_cores=NC, num_subcores=NS)


@functools.partial(
    pl.kernel,
    out_type=jax.ShapeDtypeStruct((NC, NP, D), jnp.float32),
    mesh=_mesh,
    scratch_types=[
        pltpu.VMEM((CHUNK,), jnp.int32),
        pltpu.VMEM((CHUNK, D), jnp.float32),
        pltpu.VMEM_SHARED((NP, D), jnp.float32),
    ],
)
def _deg_kernel(dst_hbm, ones_hbm, z_hbm, out_hbm, dst_buf, ones_buf, acc_sh):
    c = lax.axis_index("c")
    s = lax.axis_index("s")
    wid = c * NS + s
    pltpu.sync_copy(ones_hbm, ones_buf)
    pltpu.sync_copy(z_hbm, acc_sh.at[pl.ds(s * RPT, RPT)])
    plsc.subcore_barrier()
    base = wid * EPT

    def body(i, carry):
        pltpu.sync_copy(dst_hbm.at[pl.ds(base + i * CHUNK, CHUNK)], dst_buf)
        pltpu.sync_copy(ones_buf, acc_sh.at[dst_buf], add=True)
        return carry

    lax.fori_loop(0, NCHUNK, body, 0)
    plsc.subcore_barrier()
    pltpu.sync_copy(acc_sh.at[pl.ds(s * RPT, RPT)],
                    out_hbm.at[c, pl.ds(s * RPT, RPT)])


@functools.partial(
    pl.kernel,
    out_type=jax.ShapeDtypeStruct((NC, NP, D), jnp.float32),
    mesh=_mesh,
    scratch_types=[
        pltpu.VMEM((CHUNK,), jnp.int32),
        pltpu.VMEM((CHUNK,), jnp.int32),
        pltpu.VMEM((CHUNK, D), jnp.float32),
        pltpu.VMEM_SHARED((NP, D), jnp.float32),
        pltpu.SemaphoreType.DMA,
    ],
)
def _agg_kernel(hp_hbm, src_hbm, dst_hbm, z_hbm, out_hbm,
                src_buf, dst_buf, rows_buf, acc_sh, sem):
    c = lax.axis_index("c")
    s = lax.axis_index("s")
    wid = c * NS + s
    pltpu.sync_copy(z_hbm, acc_sh.at[pl.ds(s * RPT, RPT)])
    plsc.subcore_barrier()
    base = wid * EPT

    def body(i, carry):
        off = base + i * CHUNK
        pltpu.sync_copy(src_hbm.at[pl.ds(off, CHUNK)], src_buf)
        pltpu.sync_copy(dst_hbm.at[pl.ds(off, CHUNK)], dst_buf)
        pltpu.async_copy(hp_hbm.at[src_buf], rows_buf, sem).wait()
        pltpu.sync_copy(rows_buf, acc_sh.at[dst_buf], add=True)
        return carry

    lax.fori_loop(0, NCHUNK, body, 0)
    plsc.subcore_barrier()
    pltpu.sync_copy(acc_sh.at[pl.ds(s * RPT, RPT)],
                    out_hbm.at[c, pl.ds(s * RPT, RPT)])


BR = 1280


def _k1_body(x_ref, w_ref, d0_ref, d1_ref, hp_ref, dis_ref):
    deg = d0_ref[...] + d1_ref[...] + 1.0
    dis = lax.rsqrt(deg)
    dis_ref[...] = dis
    h = jnp.dot(x_ref[...], w_ref[...], preferred_element_type=jnp.float32)
    hp_ref[...] = h * dis


def _pre_body(a0_ref, a1_ref, hp_ref, dis_ref, b_ref, out_ref):
    out_ref[...] = ((a0_ref[...] + a1_ref[...] + hp_ref[...]) * dis_ref[...]
                    + b_ref[...])


def _bn_mm_body(pre_ref, dis_ref, g_ref, be_ref, w_ref, out_ref):
    pre = pre_ref[...]
    rows = lax.broadcasted_iota(jnp.int32, (NP, 1), 0)
    mask = rows < N
    sm = jnp.sum(jnp.where(mask, pre, 0.0), axis=0, keepdims=True)
    mean = sm * (1.0 / N)
    sq = jnp.sum(jnp.where(mask, pre * pre, 0.0), axis=0, keepdims=True)
    var = sq * (1.0 / N) - mean * mean
    xb = (pre - mean) * lax.rsqrt(var + 1e-5) * g_ref[...] + be_ref[...]
    h = jnp.maximum(xb, 0.0)
    out_ref[...] = jnp.dot(h, w_ref[...],
                           preferred_element_type=jnp.float32) * dis_ref[...]


def _heads_body(a0_ref, a1_ref, hp_ref, dis_ref, b3_ref,
                w1a_ref, b1a_ref, w1b_ref, b1b_ref,
                w2a_ref, b2a_ref, w2b_ref, b2b_ref, out1_ref, out2_ref):
    h3 = ((a0_ref[...] + a1_ref[...] + hp_ref[...]) * dis_ref[...]
          + b3_ref[...])
    t1 = jnp.maximum(
        jnp.dot(h3, w1a_ref[...], preferred_element_type=jnp.float32)
        + b1a_ref[...], 0.0)
    out1_ref[...] = (jnp.dot(t1, w1b_ref[...],
                             preferred_element_type=jnp.float32) + b1b_ref[...])
    t2 = jnp.maximum(
        jnp.dot(h3, w2a_ref[...], preferred_element_type=jnp.float32)
        + b2a_ref[...], 0.0)
    out2_ref[...] = (jnp.dot(t2, w2b_ref[...],
                             preferred_element_type=jnp.float32) + b2b_ref[...])


def _row_spec(cols):
    return pl.BlockSpec((BR, cols), lambda i: (i, 0))


def _full_spec(r, cols):
    return pl.BlockSpec((r, cols), lambda i: (0, 0))


def _matmul_scale(x, w, d0, d1):
    return pl.pallas_call(
        _k1_body,
        grid=(NP // BR,),
        in_specs=[_row_spec(D), _full_spec(D, D), _row_spec(1), _row_spec(1)],
        out_specs=[_row_spec(D), _row_spec(1)],
        out_shape=[jax.ShapeDtypeStruct((NP, D), jnp.float32),
                   jax.ShapeDtypeStruct((NP, 1), jnp.float32)],
    )(x, w, d0, d1)


def _pre(a0, a1, hp, dis, b):
    return pl.pallas_call(
        _pre_body,
        grid=(NP // BR,),
        in_specs=[_row_spec(D), _row_spec(D), _row_spec(D), _row_spec(1),
                  _full_spec(1, D)],
        out_specs=_row_spec(D),
        out_shape=jax.ShapeDtypeStruct((NP, D), jnp.float32),
    )(a0, a1, hp, dis, b)


def _bn_relu_mm_scale(pre, dis, g, be, w):
    return pl.pallas_call(
        _bn_mm_body,
        out_shape=jax.ShapeDtypeStruct((NP, D), jnp.float32),
    )(pre, dis, g, be, w)


def _heads(a0, a1, hp, dis, b3, w1a, b1a, w1b, b1b, w2a, b2a, w2b, b2b):
    l1 = w1b.shape[1]
    l2 = w2b.shape[1]
    return pl.pallas_call(
        _heads_body,
        grid=(NP // BR,),
        in_specs=[_row_spec(D), _row_spec(D), _row_spec(D), _row_spec(1),
                  _full_spec(1, D),
                  _full_spec(D, 64), _full_spec(1, 64),
                  _full_spec(64, l1), _full_spec(1, l1),
                  _full_spec(D, 64), _full_spec(1, 64),
                  _full_spec(64, l2), _full_spec(1, l2)],
        out_specs=[_row_spec(l1), _row_spec(l2)],
        out_shape=[jax.ShapeDtypeStruct((NP, l1), jnp.float32),
                   jax.ShapeDtypeStruct((NP, l2), jnp.float32)],
    )(a0, a1, hp, dis, b3, w1a, b1a, w1b, b1b, w2a, b2a, w2b, b2b)


def kernel(x, edge_index, W1, b1, W2, b2, W3, b3, g1, be1, g2, be2,
           Wc1a, bc1a, Wc1b, bc1b, Wc2a, bc2a, Wc2b, bc2b):
    src = edge_index[0]
    dst = edge_index[1]
    x_pad = jnp.pad(x, ((0, NP - N), (0, 0)))
    zeros_rows = jnp.zeros((RPT, D), jnp.float32)
    ones_rows = jnp.ones((CHUNK, D), jnp.float32)

    deg_parts = _deg_kernel(dst, ones_rows, zeros_rows)
    d0 = deg_parts[0, :, 0:1]
    d1 = deg_parts[1, :, 0:1]

    h1p, dis = _matmul_scale(x_pad, W1, d0, d1)

    a1p = _agg_kernel(h1p, src, dst, zeros_rows)
    pre1 = _pre(a1p[0], a1p[1], h1p, dis, b1.reshape(1, D))
    h2p = _bn_relu_mm_scale(pre1, dis, g1.reshape(1, D), be1.reshape(1, D), W2)

    a2p = _agg_kernel(h2p, src, dst, zeros_rows)
    pre2 = _pre(a2p[0], a2p[1], h2p, dis, b2.reshape(1, D))
    h3p = _bn_relu_mm_scale(pre2, dis, g2.reshape(1, D), be2.reshape(1, D), W3)

    a3p = _agg_kernel(h3p, src, dst, zeros_rows)
    out1, out2 = _heads(
        a3p[0], a3p[1], h3p, dis, b3.reshape(1, D),
        Wc1a, bc1a.reshape(1, -1), Wc1b, bc1b.reshape(1, -1),
        Wc2a, bc2a.reshape(1, -1), Wc2b, bc2b.reshape(1, -1))
    return (out1[:N], out2[:N])
